# baseline (speedup 1.0000x reference)
import functools

import jax
import jax.numpy as jnp
from jax import lax
from jax.experimental import pallas as pl
from jax.experimental.pallas import tpu as pltpu
from jax.experimental.pallas import tpu_sc as plsc

N = 10000
E = 320000
D = 128
G = 64

NC = 2
NS = 16
NW = NC * NS
CH = 128
CPT = 80
NPH = 2
CPP = CPT // NPH
E_PAD = NW * CPT * CH
N_PAD = 10240
RPT = N_PAD // NS

BLK = 1024


_MESH = plsc.VectorSubcoreMesh(core_axis_name="c", subcore_axis_name="s")


@functools.partial(
    pl.kernel,
    out_type=jax.ShapeDtypeStruct((NC, N_PAD, D), jnp.float32),
    mesh=_MESH,
    scratch_types=[
        pltpu.VMEM((CPP, CH), jnp.int32),
        pltpu.VMEM((CPP, CH), jnp.int32),
        pltpu.VMEM((CH, D), jnp.float32),
        pltpu.VMEM((CH, D), jnp.float32),
        pltpu.MemorySpace.VMEM_SHARED((N_PAD, D), jnp.float32),
        pltpu.SemaphoreType.DMA,
        pltpu.SemaphoreType.DMA,
        pltpu.SemaphoreType.DMA,
        pltpu.SemaphoreType.DMA,
    ],
)
def _sc_agg(src_hbm, dst_hbm, h_hbm, zeros_hbm, out_hbm,
            src_v, dst_v, buf0, buf1, agg_sh, gsem0, gsem1, ssem0, ssem1):
    c = lax.axis_index("c")
    s = lax.axis_index("s")
    wid = c * NS + s

    pltpu.sync_copy(zeros_hbm, buf0)
    for i in range(RPT // CH):
        pltpu.sync_copy(buf0, agg_sh.at[pl.ds(s * RPT + i * CH, CH)])

    plsc.subcore_barrier()

    def _wait_g(buf, sem):
        pltpu.make_async_copy(h_hbm.at[pl.ds(0, CH)], buf, sem).wait()

    def _wait_s(buf, sem):
        pltpu.make_async_copy(buf, agg_sh.at[dst_v.at[0]], sem).wait()

    for ph in range(NPH):
        pltpu.sync_copy(src_hbm.at[wid, pl.ds(ph * CPP, CPP)], src_v)
        pltpu.sync_copy(dst_hbm.at[wid, pl.ds(ph * CPP, CPP)], dst_v)
        pltpu.async_copy(h_hbm.at[src_v.at[0]], buf0, gsem0)
        pltpu.async_copy(h_hbm.at[src_v.at[1]], buf1, gsem1)

        def pair(j, carry):
            _wait_g(buf0, gsem0)
            pltpu.async_copy(buf0, agg_sh.at[dst_v.at[j]], ssem0, add=True)
            _wait_g(buf1, gsem1)
            pltpu.async_copy(buf1, agg_sh.at[dst_v.at[j + 1]], ssem1, add=True)

            @pl.when(j + 2 < CPP)
            def _():
                _wait_s(buf0, ssem0)
                pltpu.async_copy(h_hbm.at[src_v.at[j + 2]], buf0, gsem0)
                _wait_s(buf1, ssem1)
                pltpu.async_copy(h_hbm.at[src_v.at[j + 3]], buf1, gsem1)

            return carry

        lax.fori_loop(0, CPP // 2, lambda i, carry: pair(2 * i, carry), 0)
        _wait_s(buf0, ssem0)
        _wait_s(buf1, ssem1)

    plsc.subcore_barrier()
    pltpu.sync_copy(agg_sh.at[pl.ds(s * RPT, RPT)],
                    out_hbm.at[c, pl.ds(s * RPT, RPT)])


def _tc_layer_body(h_ref, a_ref, w_ref, b_ref, o_ref):
    z = h_ref[...] + a_ref[0] + a_ref[1]
    y = jnp.dot(z, w_ref[...], preferred_element_type=jnp.float32) + b_ref[...]
    o_ref[...] = jnp.maximum(y, 0.0)


def _tc_layer(h, a, w, b):
    grid = N_PAD // BLK
    return pl.pallas_call(
        _tc_layer_body,
        grid=(grid,),
        in_specs=[
            pl.BlockSpec((BLK, D), lambda i: (i, 0)),
            pl.BlockSpec((NC, BLK, D), lambda i: (0, i, 0)),
            pl.BlockSpec((D, D), lambda i: (0, 0)),
            pl.BlockSpec((1, D), lambda i: (0, 0)),
        ],
        out_specs=pl.BlockSpec((BLK, D), lambda i: (i, 0)),
        out_shape=jax.ShapeDtypeStruct((N_PAD, D), jnp.float32),
    )(h, a, w, b.reshape(1, D))


def _tc_layer3_pool_body(h_ref, a_ref, w_ref, b_ref, batch_ref,
                         wout_ref, bout_ref, o_ref, acc_ref, cnt_ref):
    i = pl.program_id(0)

    @pl.when(i == 0)
    def _():
        acc_ref[...] = jnp.zeros_like(acc_ref)
        cnt_ref[...] = jnp.zeros_like(cnt_ref)

    z = h_ref[...] + a_ref[0] + a_ref[1]
    h3 = jnp.maximum(
        jnp.dot(z, w_ref[...], preferred_element_type=jnp.float32) + b_ref[...],
        0.0)
    bb = batch_ref[0, 0, :]
    m = (lax.broadcasted_iota(jnp.int32, (G, BLK), 0) == bb[None, :]
         ).astype(jnp.float32)
    acc_ref[...] += jnp.dot(m, h3, preferred_element_type=jnp.float32)
    cnt_ref[...] += jnp.sum(m, axis=1, keepdims=True)

    @pl.when(i == pl.num_programs(0) - 1)
    def _():
        pooled = acc_ref[...] / jnp.clip(cnt_ref[...], 1.0, None)
        o_ref[...] = (jnp.dot(pooled, wout_ref[...],
                              preferred_element_type=jnp.float32)
                      + bout_ref[...])


def _tc_layer3_pool(h, a, w, b, batch_r, wout, bout):
    grid = N_PAD // BLK
    return pl.pallas_call(
        _tc_layer3_pool_body,
        grid=(grid,),
        in_specs=[
            pl.BlockSpec((BLK, D), lambda i: (i, 0)),
            pl.BlockSpec((NC, BLK, D), lambda i: (0, i, 0)),
            pl.BlockSpec((D, D), lambda i: (0, 0)),
            pl.BlockSpec((1, D), lambda i: (0, 0)),
            pl.BlockSpec((1, 1, BLK), lambda i: (i, 0, 0)),
            pl.BlockSpec((D, D), lambda i: (0, 0)),
            pl.BlockSpec((1, D), lambda i: (0, 0)),
        ],
        out_specs=pl.BlockSpec((G, D), lambda i: (0, 0)),
        out_shape=jax.ShapeDtypeStruct((G, D), jnp.float32),
        scratch_shapes=[
            pltpu.VMEM((G, D), jnp.float32),
            pltpu.VMEM((G, D), jnp.float32),
        ],
    )(h, a, w, b.reshape(1, D), batch_r, wout, bout.reshape(1, D))


def kernel(x, edge_index, batch, W1, b1, W2, b2, W3, b3, Wout, bout):
    src = edge_index[0]
    dst = edge_index[1]
    pad_e = E_PAD - E
    src_r = jnp.concatenate(
        [src, jnp.full((pad_e,), N, jnp.int32)]).reshape(NW, CPT, CH)
    dst_r = jnp.concatenate(
        [dst, jnp.full((pad_e,), N, jnp.int32)]).reshape(NW, CPT, CH)
    x_pad = jnp.concatenate(
        [x, jnp.zeros((N_PAD - N, D), jnp.float32)], axis=0)
    batch_r = jnp.concatenate(
        [batch, jnp.full((N_PAD - N,), G, jnp.int32)]
    ).reshape(N_PAD // BLK, 1, BLK)
    zeros_chunk = jnp.zeros((CH, D), jnp.float32)

    h = x_pad
    for w, b in ((W1, b1), (W2, b2)):
        a = _sc_agg(src_r, dst_r, h, zeros_chunk)
        h = _tc_layer(h, a, w, b)
    a = _sc_agg(src_r, dst_r, h, zeros_chunk)
    return _tc_layer3_pool(h, a, W3, b3, batch_r, Wout, bout)


# speedup vs baseline: 1.0297x; 1.0297x over previous
import functools

import jax
import jax.numpy as jnp
from jax import lax
from jax.experimental import pallas as pl
from jax.experimental.pallas import tpu as pltpu
from jax.experimental.pallas import tpu_sc as plsc

N = 10000
E = 320000
D = 128
G = 64

NC = 2
NS = 16
NW = NC * NS
CH = 128
CPT = 80
NPH = 2
CPP = CPT // NPH
E_PAD = NW * CPT * CH
N_PAD = 10240
RPT = N_PAD // NS

BLK = 1024


_MESH = plsc.VectorSubcoreMesh(core_axis_name="c", subcore_axis_name="s")


@functools.partial(
    pl.kernel,
    out_type=jax.ShapeDtypeStruct((NC, N_PAD, D), jnp.float32),
    mesh=_MESH,
    scratch_types=[
        pltpu.VMEM((CPP, CH), jnp.int32),
        pltpu.VMEM((CPP, CH), jnp.int32),
        pltpu.VMEM((CH, D), jnp.float32),
        pltpu.VMEM((CH, D), jnp.float32),
        pltpu.MemorySpace.VMEM_SHARED((N_PAD, D), jnp.float32),
        pltpu.SemaphoreType.DMA,
        pltpu.SemaphoreType.DMA,
    ],
)
def _sc_agg(src_hbm, dst_hbm, h_hbm, zeros_hbm, out_hbm,
            src_v, dst_v, buf0, buf1, agg_sh, sem0, sem1):
    c = lax.axis_index("c")
    s = lax.axis_index("s")
    wid = c * NS + s

    pltpu.sync_copy(zeros_hbm, buf0)
    for i in range(RPT // CH):
        pltpu.sync_copy(buf0, agg_sh.at[pl.ds(s * RPT + i * CH, CH)])
    plsc.subcore_barrier()

    for ph in range(NPH):
        pltpu.sync_copy(src_hbm.at[wid, pl.ds(ph * CPP, CPP)], src_v)
        pltpu.sync_copy(dst_hbm.at[wid, pl.ds(ph * CPP, CPP)], dst_v)
        pltpu.async_copy(h_hbm.at[src_v.at[0]], buf0, sem0)

        def pair(j, carry):
            pltpu.async_copy(h_hbm.at[src_v.at[j + 1]], buf1, sem1)
            pltpu.make_async_copy(h_hbm.at[pl.ds(0, CH)], buf0, sem0).wait()
            pltpu.sync_copy(buf0, agg_sh.at[dst_v.at[j]], add=True)

            @pl.when(j + 2 < CPP)
            def _():
                pltpu.async_copy(h_hbm.at[src_v.at[j + 2]], buf0, sem0)

            pltpu.make_async_copy(h_hbm.at[pl.ds(0, CH)], buf1, sem1).wait()
            pltpu.sync_copy(buf1, agg_sh.at[dst_v.at[j + 1]], add=True)
            return carry

        lax.fori_loop(0, CPP // 2, lambda i, carry: pair(2 * i, carry), 0)

    plsc.subcore_barrier()
    pltpu.sync_copy(agg_sh.at[pl.ds(s * RPT, RPT)],
                    out_hbm.at[c, pl.ds(s * RPT, RPT)])


def _tc_layer_body(h_ref, a_ref, w_ref, b_ref, o_ref):
    z = h_ref[...] + a_ref[0] + a_ref[1]
    y = jnp.dot(z, w_ref[...], preferred_element_type=jnp.float32) + b_ref[...]
    o_ref[...] = jnp.maximum(y, 0.0)


def _tc_layer(h, a, w, b):
    grid = N_PAD // BLK
    return pl.pallas_call(
        _tc_layer_body,
        grid=(grid,),
        in_specs=[
            pl.BlockSpec((BLK, D), lambda i: (i, 0)),
            pl.BlockSpec((NC, BLK, D), lambda i: (0, i, 0)),
            pl.BlockSpec((D, D), lambda i: (0, 0)),
            pl.BlockSpec((1, D), lambda i: (0, 0)),
        ],
        out_specs=pl.BlockSpec((BLK, D), lambda i: (i, 0)),
        out_shape=jax.ShapeDtypeStruct((N_PAD, D), jnp.float32),
    )(h, a, w, b.reshape(1, D))


def _tc_layer3_pool_body(h_ref, a_ref, w_ref, b_ref, batch_ref,
                         wout_ref, bout_ref, o_ref, acc_ref, cnt_ref):
    i = pl.program_id(0)

    @pl.when(i == 0)
    def _():
        acc_ref[...] = jnp.zeros_like(acc_ref)
        cnt_ref[...] = jnp.zeros_like(cnt_ref)

    z = h_ref[...] + a_ref[0] + a_ref[1]
    h3 = jnp.maximum(
        jnp.dot(z, w_ref[...], preferred_element_type=jnp.float32) + b_ref[...],
        0.0)
    bb = batch_ref[0, 0, :]
    m = (lax.broadcasted_iota(jnp.int32, (G, BLK), 0) == bb[None, :]
         ).astype(jnp.float32)
    acc_ref[...] += jnp.dot(m, h3, preferred_element_type=jnp.float32)
    cnt_ref[...] += jnp.sum(m, axis=1, keepdims=True)

    @pl.when(i == pl.num_programs(0) - 1)
    def _():
        pooled = acc_ref[...] / jnp.clip(cnt_ref[...], 1.0, None)
        o_ref[...] = (jnp.dot(pooled, wout_ref[...],
                              preferred_element_type=jnp.float32)
                      + bout_ref[...])


def _tc_layer3_pool(h, a, w, b, batch_r, wout, bout):
    grid = N_PAD // BLK
    return pl.pallas_call(
        _tc_layer3_pool_body,
        grid=(grid,),
        in_specs=[
            pl.BlockSpec((BLK, D), lambda i: (i, 0)),
            pl.BlockSpec((NC, BLK, D), lambda i: (0, i, 0)),
            pl.BlockSpec((D, D), lambda i: (0, 0)),
            pl.BlockSpec((1, D), lambda i: (0, 0)),
            pl.BlockSpec((1, 1, BLK), lambda i: (i, 0, 0)),
            pl.BlockSpec((D, D), lambda i: (0, 0)),
            pl.BlockSpec((1, D), lambda i: (0, 0)),
        ],
        out_specs=pl.BlockSpec((G, D), lambda i: (0, 0)),
        out_shape=jax.ShapeDtypeStruct((G, D), jnp.float32),
        scratch_shapes=[
            pltpu.VMEM((G, D), jnp.float32),
            pltpu.VMEM((G, D), jnp.float32),
        ],
    )(h, a, w, b.reshape(1, D), batch_r, wout, bout.reshape(1, D))


def kernel(x, edge_index, batch, W1, b1, W2, b2, W3, b3, Wout, bout):
    src = edge_index[0]
    dst = edge_index[1]
    pad_e = E_PAD - E
    src_r = jnp.concatenate(
        [src, jnp.full((pad_e,), N, jnp.int32)]).reshape(NW, CPT, CH)
    dst_r = jnp.concatenate(
        [dst, jnp.full((pad_e,), N, jnp.int32)]).reshape(NW, CPT, CH)
    x_pad = jnp.concatenate(
        [x, jnp.zeros((N_PAD - N, D), jnp.float32)], axis=0)
    batch_r = jnp.concatenate(
        [batch, jnp.full((N_PAD - N,), G, jnp.int32)]
    ).reshape(N_PAD // BLK, 1, BLK)
    zeros_chunk = jnp.zeros((CH, D), jnp.float32)

    h = x_pad
    for w, b in ((W1, b1), (W2, b2)):
        a = _sc_agg(src_r, dst_r, h, zeros_chunk)
        h = _tc_layer(h, a, w, b)
    a = _sc_agg(src_r, dst_r, h, zeros_chunk)
    return _tc_layer3_pool(h, a, W3, b3, batch_r, Wout, bout)
